# baseline (speedup 1.0000x reference)
import jax
import jax.numpy as jnp
from jax import lax
from jax.experimental import pallas as pl
from jax.experimental.pallas import tpu as pltpu
from jax.experimental.pallas import tpu_sc as plsc

_GATHER_WINDOW = 128
_VOCAB_TILE = 2048
_TPOSE_TILE = 2048


def _tc_transpose(tableT):
    d, v = tableT.shape

    def tr_kernel(t_ref, o_ref):
        o_ref[...] = jnp.transpose(t_ref[...])

    return pl.pallas_call(
        tr_kernel,
        grid=(pl.cdiv(v, _TPOSE_TILE),),
        in_specs=[pl.BlockSpec((d, _TPOSE_TILE), lambda i: (0, i))],
        out_specs=pl.BlockSpec((_TPOSE_TILE, d), lambda i: (i, 0)),
        out_shape=jax.ShapeDtypeStruct((v, d), jnp.float32),
    )(tableT)


def _sc_gather(pairs, ids_hi):
    n = ids_hi.shape[1]
    d2 = pairs.shape[1]
    mesh = plsc.VectorSubcoreMesh(core_axis_name="c", subcore_axis_name="s")

    @pl.kernel(out_type=jax.ShapeDtypeStruct((n, d2), pairs.dtype), mesh=mesh)
    def gather_kernel(pairs_hbm, ids_hbm, x_hbm):
        def body(i_vmem, o_vmem):
            pltpu.sync_copy(pairs_hbm.at[i_vmem.at[0]], o_vmem)

        pltpu.emit_pipeline(
            body,
            grid=(n // _GATHER_WINDOW,),
            in_specs=[pl.BlockSpec((1, _GATHER_WINDOW), lambda i: (0, i))],
            out_specs=[pl.BlockSpec((_GATHER_WINDOW, d2), lambda i: (i, 0))],
            core_axis_name=("c", "s"),
            dimension_semantics=(pltpu.PARALLEL,),
        )(ids_hbm, x_hbm)

    return gather_kernel(pairs, ids_hi)


def _tc_project_t(x2, parity, W, b2, seq):
    n, d2 = x2.shape
    d = d2 // 2
    bsz = n // seq
    v = W.shape[1]

    def mm_kernel(x2_ref, p_ref, w_ref, b_ref, o_ref, xs_ref):
        @pl.when((pl.program_id(0) == 0) & (pl.program_id(1) == 0))
        def _():
            p = p_ref[...]
            xs_ref[...] = (
                x2_ref[:, :d] * (1 - p) + x2_ref[:, d:] * p
            ).astype(jnp.bfloat16)

        l = pl.program_id(1)
        xs = xs_ref[pl.ds(l * bsz, bsz), :]
        wt = w_ref[...].astype(jnp.bfloat16)
        acc = lax.dot_general(
            wt, xs,
            dimension_numbers=(((0,), (1,)), ((), ())),
            preferred_element_type=jnp.float32,
        )
        bias = lax.dot_general(
            b_ref[...], jnp.ones((1, bsz), jnp.float32),
            dimension_numbers=(((0,), (0,)), ((), ())),
            preferred_element_type=jnp.float32,
        )
        o_ref[0] = acc + bias

    return pl.pallas_call(
        mm_kernel,
        grid=(pl.cdiv(v, _VOCAB_TILE), seq),
        in_specs=[
            pl.BlockSpec((n, d2), lambda i, l: (0, 0)),
            pl.BlockSpec((n, 1), lambda i, l: (0, 0)),
            pl.BlockSpec((d, _VOCAB_TILE), lambda i, l: (0, i)),
            pl.BlockSpec((1, _VOCAB_TILE), lambda i, l: (0, i)),
        ],
        out_specs=pl.BlockSpec((1, _VOCAB_TILE, bsz), lambda i, l: (l, i, 0)),
        out_shape=jax.ShapeDtypeStruct((seq, v, bsz), jnp.float32),
        scratch_shapes=[pltpu.VMEM((n, d), jnp.bfloat16)],
    )(x2, parity, W, b2)


def kernel(input_ids, table, W, b):
    bsz, seq = input_ids.shape
    v, d = table.shape
    n = bsz * seq
    ids_perm = input_ids.T.reshape(1, n).astype(jnp.int32)
    ids_hi = ids_perm // 2
    parity = (ids_perm & 1).reshape(n, 1).astype(jnp.float32)
    table_rm = _tc_transpose(table.T)
    pairs = table_rm.reshape(v // 2, 2 * d)
    x2 = _sc_gather(pairs, ids_hi)
    logits_t = _tc_project_t(x2, parity, W, b.reshape(1, -1), seq)
    return logits_t.transpose(2, 0, 1)


# speedup vs baseline: 1.1236x; 1.1236x over previous
import jax
import jax.numpy as jnp
from jax import lax
from jax.experimental import pallas as pl
from jax.experimental.pallas import tpu as pltpu
from jax.experimental.pallas import tpu_sc as plsc

_GATHER_WINDOW = 128
_VOCAB_TILE = 2048
_TPOSE_TILE = 2048


def _tc_transpose(tableT):
    d, v = tableT.shape

    def tr_kernel(t_ref, o_ref):
        o_ref[...] = jnp.transpose(t_ref[...])

    return pl.pallas_call(
        tr_kernel,
        grid=(pl.cdiv(v, _TPOSE_TILE),),
        in_specs=[pl.BlockSpec((d, _TPOSE_TILE), lambda i: (0, i))],
        out_specs=pl.BlockSpec((_TPOSE_TILE, d), lambda i: (i, 0)),
        out_shape=jax.ShapeDtypeStruct((v, d), jnp.float32),
    )(tableT)


def _sc_gather(pairs, ids_hi):
    n = ids_hi.shape[1]
    d2 = pairs.shape[1]
    mesh = plsc.VectorSubcoreMesh(core_axis_name="c", subcore_axis_name="s")

    @pl.kernel(out_type=jax.ShapeDtypeStruct((n, d2), pairs.dtype), mesh=mesh)
    def gather_kernel(pairs_hbm, ids_hbm, x_hbm):
        def body(i_vmem, o_vmem):
            pltpu.sync_copy(pairs_hbm.at[i_vmem.at[0]], o_vmem)

        pltpu.emit_pipeline(
            body,
            grid=(n // _GATHER_WINDOW,),
            in_specs=[pl.BlockSpec((1, _GATHER_WINDOW), lambda i: (0, i))],
            out_specs=[pl.BlockSpec((_GATHER_WINDOW, d2), lambda i: (i, 0))],
            core_axis_name=("c", "s"),
            dimension_semantics=(pltpu.PARALLEL,),
        )(ids_hbm, x_hbm)

    return gather_kernel(pairs, ids_hi)


def _tc_project_t(x2, parity, W, b2, seq):
    n, d2 = x2.shape
    d = d2 // 2
    bsz = n // seq
    v = W.shape[1]

    def mm_kernel(x2_ref, p_ref, w_ref, b_ref, o_ref, xs_ref):
        @pl.when((pl.program_id(0) == 0) & (pl.program_id(1) == 0))
        def _():
            p = p_ref[...]
            xs_ref[...] = (
                x2_ref[:, :d] * (1 - p) + x2_ref[:, d:] * p
            ).astype(jnp.bfloat16)

        l = pl.program_id(1)
        xs = xs_ref[pl.ds(l * bsz, bsz), :]
        wt = w_ref[...].astype(jnp.bfloat16)
        acc = jnp.dot(xs, wt, preferred_element_type=jnp.float32)
        o_ref[0] = jnp.transpose(acc + b_ref[...])

    return pl.pallas_call(
        mm_kernel,
        grid=(pl.cdiv(v, _VOCAB_TILE), seq),
        in_specs=[
            pl.BlockSpec((n, d2), lambda i, l: (0, 0)),
            pl.BlockSpec((n, 1), lambda i, l: (0, 0)),
            pl.BlockSpec((d, _VOCAB_TILE), lambda i, l: (0, i)),
            pl.BlockSpec((1, _VOCAB_TILE), lambda i, l: (0, i)),
        ],
        out_specs=pl.BlockSpec((1, _VOCAB_TILE, bsz), lambda i, l: (l, i, 0)),
        out_shape=jax.ShapeDtypeStruct((seq, v, bsz), jnp.float32),
        scratch_shapes=[pltpu.VMEM((n, d), jnp.bfloat16)],
    )(x2, parity, W, b2)


def kernel(input_ids, table, W, b):
    bsz, seq = input_ids.shape
    v, d = table.shape
    n = bsz * seq
    ids_perm = input_ids.T.reshape(1, n).astype(jnp.int32)
    ids_hi = ids_perm // 2
    parity = (ids_perm & 1).reshape(n, 1).astype(jnp.float32)
    table_rm = _tc_transpose(table.T)
    pairs = table_rm.reshape(v // 2, 2 * d)
    x2 = _sc_gather(pairs, ids_hi)
    logits_t = _tc_project_t(x2, parity, W, b.reshape(1, -1), seq)
    return logits_t.transpose(2, 0, 1)


# speedup vs baseline: 1.4674x; 1.3059x over previous
import jax
import jax.numpy as jnp
from jax import lax
from jax.experimental import pallas as pl
from jax.experimental.pallas import tpu as pltpu
from jax.experimental.pallas import tpu_sc as plsc

_GATHER_WINDOW = 128
_VOCAB_TILE = 2048
_TPOSE_TILE = 2048


def _tc_transpose(tableT):
    d, v = tableT.shape

    def tr_kernel(t_ref, o_ref):
        o_ref[...] = jnp.transpose(t_ref[...])

    return pl.pallas_call(
        tr_kernel,
        grid=(pl.cdiv(v, _TPOSE_TILE),),
        in_specs=[pl.BlockSpec((d, _TPOSE_TILE), lambda i: (0, i))],
        out_specs=pl.BlockSpec((_TPOSE_TILE, d), lambda i: (i, 0)),
        out_shape=jax.ShapeDtypeStruct((v, d), jnp.float32),
    )(tableT)


def _sc_gather(pairs, ids_hi):
    n = ids_hi.shape[1]
    d2 = pairs.shape[1]
    mesh = plsc.VectorSubcoreMesh(core_axis_name="c", subcore_axis_name="s")

    @pl.kernel(out_type=jax.ShapeDtypeStruct((n, d2), pairs.dtype), mesh=mesh)
    def gather_kernel(pairs_hbm, ids_hbm, x_hbm):
        def body(i_vmem, o_vmem):
            pltpu.sync_copy(pairs_hbm.at[i_vmem.at[0]], o_vmem)

        pltpu.emit_pipeline(
            body,
            grid=(n // _GATHER_WINDOW,),
            in_specs=[pl.BlockSpec((1, _GATHER_WINDOW), lambda i: (0, i))],
            out_specs=[pl.BlockSpec((_GATHER_WINDOW, d2), lambda i: (i, 0))],
            core_axis_name=("c", "s"),
            dimension_semantics=(pltpu.PARALLEL,),
        )(ids_hbm, x_hbm)

    return gather_kernel(pairs, ids_hi)


def _tc_project_t(x2, parity, W, b2, seq):
    n, d2 = x2.shape
    d = d2 // 2
    bsz = n // seq
    v = W.shape[1]

    ka = 72

    def mm_kernel(x2_ref, p_ref, w_ref, b_ref, o_ref, xst_ref, wa_ref):
        @pl.when((pl.program_id(0) == 0) & (pl.program_id(1) == 0))
        def _():
            p = p_ref[...]
            xs = (x2_ref[:, :d] * (1 - p) + x2_ref[:, d:] * p).astype(
                jnp.bfloat16
            )
            xst_ref[:d, :] = jnp.transpose(xs)
            xst_ref[d:d + 1, :] = jnp.ones((1, n), jnp.bfloat16)
            xst_ref[d + 1:, :] = jnp.zeros((ka - d - 1, n), jnp.bfloat16)
            wa_ref[d + 1:, :] = jnp.zeros((ka - d - 1, _VOCAB_TILE), jnp.bfloat16)

        l2 = pl.program_id(1)
        wa_ref[:d, :] = w_ref[...].astype(jnp.bfloat16)
        wa_ref[d:d + 1, :] = b_ref[...].astype(jnp.bfloat16)
        xst2 = xst_ref[:, pl.ds(l2 * 2 * bsz, 2 * bsz)]
        acc = lax.dot_general(
            wa_ref[...], xst2,
            dimension_numbers=(((0,), (0,)), ((), ())),
            preferred_element_type=jnp.float32,
        )
        o_ref[0] = acc[:, :bsz]
        o_ref[1] = acc[:, bsz:]

    return pl.pallas_call(
        mm_kernel,
        grid=(pl.cdiv(v, _VOCAB_TILE), seq // 2),
        in_specs=[
            pl.BlockSpec((n, d2), lambda i, l2: (0, 0)),
            pl.BlockSpec((n, 1), lambda i, l2: (0, 0)),
            pl.BlockSpec((d, _VOCAB_TILE), lambda i, l2: (0, i)),
            pl.BlockSpec((1, _VOCAB_TILE), lambda i, l2: (0, i)),
        ],
        out_specs=pl.BlockSpec((2, _VOCAB_TILE, bsz), lambda i, l2: (l2, i, 0)),
        out_shape=jax.ShapeDtypeStruct((seq, v, bsz), jnp.float32),
        scratch_shapes=[
            pltpu.VMEM((ka, n), jnp.bfloat16),
            pltpu.VMEM((ka, _VOCAB_TILE), jnp.bfloat16),
        ],
    )(x2, parity, W, b2)


def kernel(input_ids, table, W, b):
    bsz, seq = input_ids.shape
    v, d = table.shape
    n = bsz * seq
    ids_perm = input_ids.T.reshape(1, n).astype(jnp.int32)
    ids_hi = ids_perm // 2
    parity = (ids_perm & 1).reshape(n, 1).astype(jnp.float32)
    table_rm = _tc_transpose(table.T)
    pairs = table_rm.reshape(v // 2, 2 * d)
    x2 = _sc_gather(pairs, ids_hi)
    logits_t = _tc_project_t(x2, parity, W, b.reshape(1, -1), seq)
    return logits_t.transpose(2, 0, 1)


# speedup vs baseline: 1.7328x; 1.1809x over previous
import jax
import jax.numpy as jnp
from jax import lax
from jax.experimental import pallas as pl
from jax.experimental.pallas import tpu as pltpu
from jax.experimental.pallas import tpu_sc as plsc

_GATHER_WINDOW = 128
_VOCAB_TILE = 4096
_TPOSE_TILE = 2048


def _tc_transpose(tableT):
    d, v = tableT.shape

    def tr_kernel(t_ref, o_ref):
        o_ref[...] = jnp.transpose(t_ref[...])

    return pl.pallas_call(
        tr_kernel,
        grid=(pl.cdiv(v, _TPOSE_TILE),),
        in_specs=[pl.BlockSpec((d, _TPOSE_TILE), lambda i: (0, i))],
        out_specs=pl.BlockSpec((_TPOSE_TILE, d), lambda i: (i, 0)),
        out_shape=jax.ShapeDtypeStruct((v, d), jnp.float32),
    )(tableT)


def _sc_gather(pairs, ids_hi):
    n = ids_hi.shape[1]
    d2 = pairs.shape[1]
    mesh = plsc.VectorSubcoreMesh(core_axis_name="c", subcore_axis_name="s")

    @pl.kernel(out_type=jax.ShapeDtypeStruct((n, d2), pairs.dtype), mesh=mesh)
    def gather_kernel(pairs_hbm, ids_hbm, x_hbm):
        def body(i_vmem, o_vmem):
            pltpu.sync_copy(pairs_hbm.at[i_vmem.at[0]], o_vmem)

        pltpu.emit_pipeline(
            body,
            grid=(n // _GATHER_WINDOW,),
            in_specs=[pl.BlockSpec((1, _GATHER_WINDOW), lambda i: (0, i))],
            out_specs=[pl.BlockSpec((_GATHER_WINDOW, d2), lambda i: (i, 0))],
            core_axis_name=("c", "s"),
            dimension_semantics=(pltpu.PARALLEL,),
        )(ids_hbm, x_hbm)

    return gather_kernel(pairs, ids_hi)


def _tc_project_t(x2, parity, W, b2, seq):
    n, d2 = x2.shape
    d = d2 // 2
    bsz = n // seq
    v = W.shape[1]

    ka = 72

    def mm_kernel(x2_ref, p_ref, w_ref, b_ref, o_ref, xst_ref, wa_ref):
        @pl.when((pl.program_id(0) == 0) & (pl.program_id(1) == 0))
        def _():
            p = p_ref[...]
            xs = (x2_ref[:, :d] * (1 - p) + x2_ref[:, d:] * p).astype(
                jnp.bfloat16
            )
            xst_ref[:d, :] = jnp.transpose(xs)
            xst_ref[d:d + 1, :] = jnp.ones((1, n), jnp.bfloat16)
            xst_ref[d + 1:, :] = jnp.zeros((ka - d - 1, n), jnp.bfloat16)
            wa_ref[d + 1:, :] = jnp.zeros((ka - d - 1, _VOCAB_TILE), jnp.bfloat16)

        l2 = pl.program_id(1)
        wa_ref[:d, :] = w_ref[...]
        wa_ref[d:d + 1, :] = b_ref[...]
        xst2 = xst_ref[:, pl.ds(l2 * 2 * bsz, 2 * bsz)]
        acc = lax.dot_general(
            wa_ref[...], xst2,
            dimension_numbers=(((0,), (0,)), ((), ())),
            preferred_element_type=jnp.float32,
        )
        o_ref[0] = acc[:, :bsz]
        o_ref[1] = acc[:, bsz:]

    return pl.pallas_call(
        mm_kernel,
        grid=(pl.cdiv(v, _VOCAB_TILE), seq // 2),
        in_specs=[
            pl.BlockSpec((n, d2), lambda i, l2: (0, 0)),
            pl.BlockSpec((n, 1), lambda i, l2: (0, 0)),
            pl.BlockSpec((d, _VOCAB_TILE), lambda i, l2: (0, i)),
            pl.BlockSpec((1, _VOCAB_TILE), lambda i, l2: (0, i)),
        ],
        out_specs=pl.BlockSpec((2, _VOCAB_TILE, bsz), lambda i, l2: (l2, i, 0)),
        out_shape=jax.ShapeDtypeStruct((seq, v, bsz), jnp.float32),
        scratch_shapes=[
            pltpu.VMEM((ka, n), jnp.bfloat16),
            pltpu.VMEM((ka, _VOCAB_TILE), jnp.bfloat16),
        ],
    )(x2, parity, W, b2)


def kernel(input_ids, table, W, b):
    bsz, seq = input_ids.shape
    v, d = table.shape
    n = bsz * seq
    ids_perm = input_ids.T.reshape(1, n).astype(jnp.int32)
    ids_hi = ids_perm // 2
    parity = (ids_perm & 1).reshape(n, 1).astype(jnp.float32)
    table_rm = _tc_transpose(table.T)
    pairs = table_rm.reshape(v // 2, 2 * d)
    x2 = _sc_gather(pairs, ids_hi)
    logits_t = _tc_project_t(
        x2, parity, W.astype(jnp.bfloat16),
        b.reshape(1, -1).astype(jnp.bfloat16), seq,
    )
    return logits_t.transpose(2, 0, 1)
